# Initial kernel scaffold; baseline (speedup 1.0000x reference)
import jax
import jax.numpy as jnp
from jax.experimental import pallas as pl


def kernel(data, ent_embeds, rel_embeds, val_embeds, corrupt_idx):
    raise NotImplementedError("write your pallas kernel here")


# speedup vs baseline: 1.1527x; 1.1527x over previous
import functools

import jax
import jax.numpy as jnp
from jax import lax
from jax.experimental import pallas as pl
from jax.experimental.pallas import tpu as pltpu
from jax.experimental.pallas import tpu_sc as plsc

_NC, _NS, _L = 2, 16, 16
_NW = _NC * _NS
_B = 4096
_D = 128
_BPW = _B // _NW
_NCH = _D // _L
_MARGIN = 1.0


def _rsqrt(x):
    i = lax.bitcast_convert_type(x, jnp.int32)
    i = jnp.int32(0x5F3759DF) - (i >> 1)
    y = lax.bitcast_convert_type(i, jnp.float32)
    for _ in range(3):
        y = y * (1.5 - 0.5 * x * y * y)
    return y


def _col_sum(ref, a, lanes):
    av = jnp.full((_L,), a, jnp.int32)
    s = jnp.zeros((_L,), jnp.float32)
    for k in range(_L):
        s = s + plsc.load_gather(ref, [av, lanes, jnp.full((_L,), k, jnp.int32)])
    return s


def _sc_body(hidx, ridx, tidx, ent, rel, val, cidx, out,
             hidx_v, ridx_v, tidx_v, h_rows, r_rows, t_rows,
             cidx_v, hc_row, hc_buf, acc_t, out_buf,
             sem_h, sem_r, sem_t, sem_c):
    wid = lax.axis_index("s") * _NC + lax.axis_index("c")
    base = wid * _BPW

    pltpu.sync_copy(hidx.at[pl.ds(base, _BPW)], hidx_v)
    pltpu.sync_copy(ridx.at[pl.ds(base, _BPW)], ridx_v)
    pltpu.sync_copy(tidx.at[pl.ds(base, _BPW)], tidx_v)
    pltpu.sync_copy(cidx, cidx_v)

    cp_h = pltpu.async_copy(ent.at[hidx_v], h_rows, sem_h)
    cp_r = pltpu.async_copy(rel.at[ridx_v], r_rows, sem_r)
    cp_t = pltpu.async_copy(val.at[tidx_v], t_rows, sem_t)
    cp_c = pltpu.async_copy(ent.at[cidx_v], hc_row, sem_c)

    lanes = lax.iota(jnp.int32, _L)

    cp_c.wait()
    cc = jnp.zeros((_L,), jnp.float32)
    chunks = []
    for c in range(_NCH):
        v = hc_row[0, pl.ds(c * _L, _L)]
        chunks.append(v)
        cc = cc + v * v
    acc_t[0, 0] = cc
    cc_s = _col_sum(acc_t, 0, jnp.zeros((_L,), jnp.int32))
    inc = _rsqrt(jnp.maximum(cc_s, 1e-24))
    for c in range(_NCH):
        hc_buf[pl.ds(c * _L, _L)] = chunks[c] * inc
    ccn_s = cc_s * inc * inc

    cp_h.wait()
    cp_r.wait()
    cp_t.wait()

    def row(i, j):
        z = jnp.zeros((_L,), jnp.float32)
        hh = tt = rr = hr = ht = rt = cr = ct = z
        for c in range(_NCH):
            sl = pl.ds(c * _L, _L)
            h = h_rows[i, sl]
            r = r_rows[i, sl]
            t = t_rows[i, sl]
            hcn = hc_buf[sl]
            hh = hh + h * h
            tt = tt + t * t
            rr = rr + r * r
            hr = hr + h * r
            ht = ht + h * t
            rt = rt + r * t
            cr = cr + hcn * r
            ct = ct + hcn * t
        acc_t[0, j] = hh
        acc_t[1, j] = tt
        acc_t[2, j] = rr
        acc_t[3, j] = hr
        acc_t[4, j] = ht
        acc_t[5, j] = rt
        acc_t[6, j] = cr
        acc_t[7, j] = ct

    def group(g, carry):
        def row_j(j, carry2):
            row(g * _L + j, j)
            return carry2

        lax.fori_loop(0, _L, row_j, 0)
        hh_s = _col_sum(acc_t, 0, lanes)
        tt_s = _col_sum(acc_t, 1, lanes)
        rr_s = _col_sum(acc_t, 2, lanes)
        hr_s = _col_sum(acc_t, 3, lanes)
        ht_s = _col_sum(acc_t, 4, lanes)
        rt_s = _col_sum(acc_t, 5, lanes)
        cr_s = _col_sum(acc_t, 6, lanes)
        ct_s = _col_sum(acc_t, 7, lanes)
        inh = _rsqrt(jnp.maximum(hh_s, 1e-24))
        int_ = _rsqrt(jnp.maximum(tt_s, 1e-24))
        hn2 = hh_s * inh * inh
        tn2 = tt_s * int_ * int_
        pos2 = (hn2 + tn2 + rr_s
                + 2.0 * hr_s * inh - 2.0 * ht_s * inh * int_ - 2.0 * rt_s * int_)
        neg2 = (ccn_s + tn2 + rr_s
                + 2.0 * cr_s - 2.0 * ct_s * int_ - 2.0 * rt_s * int_)
        pos2 = jnp.maximum(pos2, 0.0)
        neg2 = jnp.maximum(neg2, 0.0)
        pos = pos2 * _rsqrt(jnp.maximum(pos2, 1e-30))
        neg = neg2 * _rsqrt(jnp.maximum(neg2, 1e-30))
        off = pl.multiple_of(g * _L, _L)
        out_buf[pl.ds(off, _L)] = pos - neg + _MARGIN
        return carry

    lax.fori_loop(0, _BPW // _L, group, 0)
    pltpu.sync_copy(out_buf, out.at[pl.ds(base, _BPW)])


_sc_kernel = functools.partial(
    pl.kernel,
    out_type=jax.ShapeDtypeStruct((_B,), jnp.float32),
    mesh=plsc.VectorSubcoreMesh(core_axis_name="c", subcore_axis_name="s"),
    compiler_params=pltpu.CompilerParams(needs_layout_passes=False),
    scratch_types=[
        pltpu.VMEM((_BPW,), jnp.int32),
        pltpu.VMEM((_BPW,), jnp.int32),
        pltpu.VMEM((_BPW,), jnp.int32),
        pltpu.VMEM((_BPW, _D), jnp.float32),
        pltpu.VMEM((_BPW, _D), jnp.float32),
        pltpu.VMEM((_BPW, _D), jnp.float32),
        pltpu.VMEM((1,), jnp.int32),
        pltpu.VMEM((1, _D), jnp.float32),
        pltpu.VMEM((_D,), jnp.float32),
        pltpu.VMEM((8, _L, _L), jnp.float32),
        pltpu.VMEM((_BPW,), jnp.float32),
        pltpu.SemaphoreType.DMA,
        pltpu.SemaphoreType.DMA,
        pltpu.SemaphoreType.DMA,
        pltpu.SemaphoreType.DMA,
    ],
)(_sc_body)


def kernel(data, ent_embeds, rel_embeds, val_embeds, corrupt_idx):
    hidx = data[:, 0]
    ridx = data[:, 1]
    tidx = data[:, 2]
    return _sc_kernel(hidx, ridx, tidx,
                      ent_embeds, rel_embeds, val_embeds, corrupt_idx)


# speedup vs baseline: 1.2885x; 1.1179x over previous
import functools

import jax
import jax.numpy as jnp
from jax import lax
from jax.experimental import pallas as pl
from jax.experimental.pallas import tpu as pltpu
from jax.experimental.pallas import tpu_sc as plsc

_NC, _NS, _L = 2, 16, 16
_NW = _NC * _NS
_B = 4096
_D = 128
_BPW = _B // _NW
_NCH = _D // _L
_MARGIN = 1.0


def _rsqrt(x):
    i = lax.bitcast_convert_type(x, jnp.int32)
    i = jnp.int32(0x5F3759DF) - (i >> 1)
    y = lax.bitcast_convert_type(i, jnp.float32)
    for _ in range(3):
        y = y * (1.5 - 0.5 * x * y * y)
    return y


_ACC = 8
_GRP = _BPW // _L
_GSTRIDE = _ACC * _L * _L


def _sc_body(hidx, ridx, tidx, ent, rel, val, cidx, out,
             hidx_v, ridx_v, tidx_v, h_rows, r_rows, t_rows,
             cidx_v, hc_row, acc_flat, out_buf,
             sem_h, sem_r, sem_t, sem_c):
    wid = lax.axis_index("s") * _NC + lax.axis_index("c")
    base = wid * _BPW

    pltpu.sync_copy(hidx.at[pl.ds(base, _BPW)], hidx_v)
    pltpu.sync_copy(ridx.at[pl.ds(base, _BPW)], ridx_v)
    pltpu.sync_copy(tidx.at[pl.ds(base, _BPW)], tidx_v)
    pltpu.sync_copy(cidx, cidx_v)

    cp_h = pltpu.async_copy(ent.at[hidx_v], h_rows, sem_h)
    cp_r = pltpu.async_copy(rel.at[ridx_v], r_rows, sem_r)
    cp_t = pltpu.async_copy(val.at[tidx_v], t_rows, sem_t)
    cp_c = pltpu.async_copy(ent.at[cidx_v], hc_row, sem_c)

    lanes = lax.iota(jnp.int32, _L)

    cp_c.wait()
    cc = jnp.zeros((_L,), jnp.float32)
    chunks = []
    for c in range(_NCH):
        v = hc_row[0, pl.ds(c * _L, _L)]
        chunks.append(v)
        cc = cc + v * v
    cco = _GRP * _GSTRIDE
    acc_flat[pl.ds(cco, _L)] = cc
    cc_s = jnp.zeros((_L,), jnp.float32)
    for k in range(_L):
        cc_s = cc_s + plsc.load_gather(
            acc_flat, [jnp.full((_L,), cco + k, jnp.int32)])
    inc = _rsqrt(jnp.maximum(cc_s, 1e-24))
    hcn = [chunks[c] * inc for c in range(_NCH)]
    ccn_s = cc_s * inc * inc

    cp_h.wait()
    cp_r.wait()
    cp_t.wait()

    lanes16 = lanes * _L

    @plsc.parallel_loop(0, _GRP)
    def _group(g):
        gb = g * _L
        ab = pl.multiple_of(g * _GSTRIDE, _GSTRIDE)
        for j in range(_L):
            i = gb + j
            z = jnp.zeros((_L,), jnp.float32)
            hh = tt = rr = hr = ht = rt = cr = ct = z
            for c in range(_NCH):
                sl = pl.ds(c * _L, _L)
                h = h_rows[i, sl]
                r = r_rows[i, sl]
                t = t_rows[i, sl]
                hh = hh + h * h
                tt = tt + t * t
                rr = rr + r * r
                hr = hr + h * r
                ht = ht + h * t
                rt = rt + r * t
                cr = cr + hcn[c] * r
                ct = ct + hcn[c] * t
            for a, acc in enumerate((hh, tt, rr, hr, ht, rt, cr, ct)):
                acc_flat[pl.ds(ab + a * (_L * _L) + j * _L, _L)] = acc
        sums = []
        for a in range(_ACC):
            idx = lanes16 + (ab + a * (_L * _L))
            s = jnp.zeros((_L,), jnp.float32)
            for k in range(_L):
                s = s + plsc.load_gather(acc_flat, [idx + k])
            sums.append(s)
        hh_s, tt_s, rr_s, hr_s, ht_s, rt_s, cr_s, ct_s = sums
        inh = _rsqrt(jnp.maximum(hh_s, 1e-24))
        int_ = _rsqrt(jnp.maximum(tt_s, 1e-24))
        hn2 = hh_s * inh * inh
        tn2 = tt_s * int_ * int_
        pos2 = (hn2 + tn2 + rr_s
                + 2.0 * hr_s * inh - 2.0 * ht_s * inh * int_ - 2.0 * rt_s * int_)
        neg2 = (ccn_s + tn2 + rr_s
                + 2.0 * cr_s - 2.0 * ct_s * int_ - 2.0 * rt_s * int_)
        pos2 = jnp.maximum(pos2, 0.0)
        neg2 = jnp.maximum(neg2, 0.0)
        pos = pos2 * _rsqrt(jnp.maximum(pos2, 1e-30))
        neg = neg2 * _rsqrt(jnp.maximum(neg2, 1e-30))
        off = pl.multiple_of(g * _L, _L)
        out_buf[pl.ds(off, _L)] = pos - neg + _MARGIN

    pltpu.sync_copy(out_buf, out.at[pl.ds(base, _BPW)])


_sc_kernel = functools.partial(
    pl.kernel,
    out_type=jax.ShapeDtypeStruct((_B,), jnp.float32),
    mesh=plsc.VectorSubcoreMesh(core_axis_name="c", subcore_axis_name="s"),
    compiler_params=pltpu.CompilerParams(needs_layout_passes=False),
    scratch_types=[
        pltpu.VMEM((_BPW,), jnp.int32),
        pltpu.VMEM((_BPW,), jnp.int32),
        pltpu.VMEM((_BPW,), jnp.int32),
        pltpu.VMEM((_BPW, _D), jnp.float32),
        pltpu.VMEM((_BPW, _D), jnp.float32),
        pltpu.VMEM((_BPW, _D), jnp.float32),
        pltpu.VMEM((1,), jnp.int32),
        pltpu.VMEM((1, _D), jnp.float32),
        pltpu.VMEM((_GRP * _GSTRIDE + _L,), jnp.float32),
        pltpu.VMEM((_BPW,), jnp.float32),
        pltpu.SemaphoreType.DMA,
        pltpu.SemaphoreType.DMA,
        pltpu.SemaphoreType.DMA,
        pltpu.SemaphoreType.DMA,
    ],
)(_sc_body)


def kernel(data, ent_embeds, rel_embeds, val_embeds, corrupt_idx):
    hidx = data[:, 0]
    ridx = data[:, 1]
    tidx = data[:, 2]
    return _sc_kernel(hidx, ridx, tidx,
                      ent_embeds, rel_embeds, val_embeds, corrupt_idx)
